# baseline (speedup 1.0000x reference)
import functools

import jax
import jax.numpy as jnp
from jax import lax
from jax.experimental import pallas as pl
from jax.experimental.pallas import tpu as pltpu
from jax.experimental.pallas import tpu_sc as plsc

_EPS = 1e-5
_NW = 32
_CH = 8192


def _sc_gather(t2T, t1T, idxT):
  f, e, v = t2T.shape
  b = idxT.shape[1]
  mesh = plsc.VectorSubcoreMesh(core_axis_name="core",
                                subcore_axis_name="subcore")

  @functools.partial(
      pl.kernel,
      out_type=(jax.ShapeDtypeStruct((f, e, b), jnp.float32),
                jax.ShapeDtypeStruct((f, b), jnp.float32)),
      mesh=mesh,
      scratch_types=[pltpu.VMEM((v,), jnp.float32),
                     pltpu.VMEM((_CH,), jnp.int32),
                     pltpu.VMEM((2, _CH), jnp.float32),
                     pltpu.SemaphoreType.DMA,
                     pltpu.SemaphoreType.DMA],
      compiler_params=pltpu.CompilerParams(use_tc_tiling_on_sc=True,
                                           needs_layout_passes=False))
  def k(t2T_hbm, t1T_hbm, idxT_hbm, o2_hbm, o1_hbm, plane, idxv, outv, s0, s1):
    w = lax.axis_index("subcore") * 2 + lax.axis_index("core")
    nchunk = b // _CH
    sems = (s0, s1)
    drain_src = o2_hbm.at[0, 0, pl.ds(0, _CH)]

    def do_plane(out_slice_fn, pf, wait_cond):
      for ci in range(nchunk):
        c = ci * _CH
        sem = sems[ci]
        pltpu.sync_copy(idxT_hbm.at[pf, pl.ds(c, _CH)], idxv)

        if wait_cond is None:
          pltpu.make_async_copy(drain_src, outv.at[ci], sem).wait()
        else:
          @pl.when(wait_cond)
          def _():
            pltpu.make_async_copy(drain_src, outv.at[ci], sem).wait()

        @pl.loop(0, _CH, step=64)
        def _(j):
          for u in range(0, 64, 16):
            outv[ci, pl.ds(j + u, 16)] = plsc.load_gather(
                plane, [idxv[pl.ds(j + u, 16)]])

        pltpu.async_copy(outv.at[ci], out_slice_fn(c), sem)

    np2 = f * e // _NW

    @pl.loop(0, np2)
    def _(pi):
      p = w + pi * _NW
      pf = p // e
      pe = p % e
      pltpu.sync_copy(t2T_hbm.at[pf, pe], plane)
      do_plane(lambda c, pf=pf, pe=pe: o2_hbm.at[pf, pe, pl.ds(c, _CH)],
               pf, pi > 0)

    @pl.when(w < f)
    def _():
      pltpu.sync_copy(t1T_hbm.at[w, 0], plane)
      do_plane(lambda c, pf=w: o1_hbm.at[pf, pl.ds(c, _CH)], w, None)

    pltpu.make_async_copy(drain_src, outv.at[0], s0).wait()
    pltpu.make_async_copy(drain_src, outv.at[1], s1).wait()

  return k(t2T, t1T, idxT)


def _stage_a_body(oT_ref, o1T_ref, dT_ref, wdl_ref, bdl_ref, w1d_ref, b1d_ref,
                  wl1_ref, bl1_ref, h1_ref, st_ref):
  i = pl.program_id(0)
  fe = oT_ref.shape[0]
  e = fe // o1T_ref.shape[0]
  oT = oT_ref[...]
  dT = dT_ref[...]
  sel = (lax.broadcasted_iota(jnp.int32, (e, fe), 0) ==
         lax.broadcasted_iota(jnp.int32, (e, fe), 1) % e).astype(jnp.float32)
  sum_embT = jax.lax.dot(sel, oT)
  order2 = 0.5 * (jnp.sum(sum_embT * sum_embT, axis=0, keepdims=True)
                  - jnp.sum(oT * oT, axis=0, keepdims=True))
  order1 = (jnp.sum(o1T_ref[...], axis=0, keepdims=True)
            + jax.lax.dot(w1d_ref[...], dT) + b1d_ref[...])
  ddT = jnp.maximum(jax.lax.dot(wdl_ref[...], dT) + bdl_ref[...], 0.0)
  dnnT = oT + ddT + order1 + order2
  h1 = jax.lax.dot(wl1_ref[...], dnnT) + bl1_ref[...]
  h1_ref[...] = h1
  st = jnp.concatenate([jnp.sum(h1, axis=1, keepdims=True),
                        jnp.sum(h1 * h1, axis=1, keepdims=True)], axis=1)

  @pl.when(i == 0)
  def _():
    st_ref[...] = st

  @pl.when(i > 0)
  def _():
    st_ref[...] = st_ref[...] + st


def _stage_mid_body(x_ref, st_in_ref, g_ref, be_ref, w_ref, bias_ref,
                    y_ref, st_ref, *, batch):
  i = pl.program_id(0)
  mean = st_in_ref[:, 0:1] / batch
  var = st_in_ref[:, 1:2] / batch - mean * mean
  a = g_ref[...] * lax.rsqrt(var + _EPS)
  c = be_ref[...] - mean * a
  x = jnp.maximum(x_ref[...] * a + c, 0.0)
  y = jax.lax.dot(w_ref[...], x) + bias_ref[...]
  y_ref[...] = y
  st = jnp.concatenate([jnp.sum(y, axis=1, keepdims=True),
                        jnp.sum(y * y, axis=1, keepdims=True)], axis=1)

  @pl.when(i == 0)
  def _():
    st_ref[...] = st

  @pl.when(i > 0)
  def _():
    st_ref[...] = st_ref[...] + st


def _stage_c_body(x_ref, st_in_ref, g_ref, be_ref, w_ref, bias_ref, out_ref,
                  *, batch):
  mean = st_in_ref[:, 0:1] / batch
  var = st_in_ref[:, 1:2] / batch - mean * mean
  a = g_ref[...] * lax.rsqrt(var + _EPS)
  c = be_ref[...] - mean * a
  x = jnp.maximum(x_ref[...] * a + c, 0.0)
  out_ref[...] = jax.lax.dot(w_ref[...], x) + bias_ref[...]


def _full(shape):
  return pl.BlockSpec(shape, lambda i: (0, 0))


def kernel(sparse, dense, order1_emb, order2_emb, W1d, b1d, Wdl, bdl, W_l1,
           b_l1, g1, be1, W_l2, b_l2, g2, be2, W_out, b_out):
  b, f = sparse.shape
  v, e = order2_emb.shape[1], order2_emb.shape[2]
  d = dense.shape[1]
  h1n, h2n = W_l1.shape[0], W_l2.shape[0]
  fe = f * e
  bb = 1024
  grid = (b // bb,)
  fl = jnp.float32

  o2T3, o1T = _sc_gather(jnp.transpose(order2_emb, (0, 2, 1)),
                         jnp.transpose(order1_emb, (0, 2, 1)), sparse.T)
  oT = o2T3.reshape(fe, b)

  h1T, st1 = pl.pallas_call(
      _stage_a_body,
      grid=grid,
      in_specs=[
          pl.BlockSpec((fe, bb), lambda i: (0, i)),
          pl.BlockSpec((f, bb), lambda i: (0, i)),
          pl.BlockSpec((d, bb), lambda i: (0, i)),
          _full((fe, d)), _full((fe, 1)), _full((1, d)), _full((1, 1)),
          _full((h1n, fe)), _full((h1n, 1)),
      ],
      out_specs=[pl.BlockSpec((h1n, bb), lambda i: (0, i)),
                 _full((h1n, 2))],
      out_shape=(jax.ShapeDtypeStruct((h1n, b), fl),
                 jax.ShapeDtypeStruct((h1n, 2), fl)),
  )(oT, o1T, dense.T, Wdl, bdl.reshape(fe, 1), W1d, b1d.reshape(1, 1),
    W_l1, b_l1.reshape(h1n, 1))

  h2T, st2 = pl.pallas_call(
      functools.partial(_stage_mid_body, batch=float(b)),
      grid=grid,
      in_specs=[
          pl.BlockSpec((h1n, bb), lambda i: (0, i)),
          _full((h1n, 2)), _full((h1n, 1)), _full((h1n, 1)),
          _full((h2n, h1n)), _full((h2n, 1)),
      ],
      out_specs=[pl.BlockSpec((h2n, bb), lambda i: (0, i)),
                 _full((h2n, 2))],
      out_shape=(jax.ShapeDtypeStruct((h2n, b), fl),
                 jax.ShapeDtypeStruct((h2n, 2), fl)),
  )(h1T, st1, g1.reshape(h1n, 1), be1.reshape(h1n, 1), W_l2,
    b_l2.reshape(h2n, 1))

  outT = pl.pallas_call(
      functools.partial(_stage_c_body, batch=float(b)),
      grid=grid,
      in_specs=[
          pl.BlockSpec((h2n, bb), lambda i: (0, i)),
          _full((h2n, 2)), _full((h2n, 1)), _full((h2n, 1)),
          _full((1, h2n)), _full((1, 1)),
      ],
      out_specs=pl.BlockSpec((1, bb), lambda i: (0, i)),
      out_shape=jax.ShapeDtypeStruct((1, b), fl),
  )(h2T, st2, g2.reshape(h2n, 1), be2.reshape(h2n, 1), W_out,
    b_out.reshape(1, 1))
  return outT.reshape(b, 1)


# speedup vs baseline: 1.2553x; 1.2553x over previous
import functools

import jax
import jax.numpy as jnp
from jax import lax
from jax.experimental import pallas as pl
from jax.experimental.pallas import tpu as pltpu
from jax.experimental.pallas import tpu_sc as plsc

_EPS = 1e-5
_NW = 32
_CH = 8192


def _sc_gather(t2T, t1T, idxT):
  f, e, v = t2T.shape
  b = idxT.shape[1]
  mesh = plsc.VectorSubcoreMesh(core_axis_name="core",
                                subcore_axis_name="subcore")

  @functools.partial(
      pl.kernel,
      out_type=(jax.ShapeDtypeStruct((f, e, b), jnp.float32),
                jax.ShapeDtypeStruct((f, b), jnp.float32)),
      mesh=mesh,
      scratch_types=[pltpu.VMEM((v,), jnp.float32),
                     pltpu.VMEM((_CH,), jnp.int32),
                     pltpu.VMEM((_CH,), jnp.float32)],
      compiler_params=pltpu.CompilerParams(use_tc_tiling_on_sc=True,
                                           needs_layout_passes=False))
  def k(t2T_hbm, t1T_hbm, idxT_hbm, o2_hbm, o1_hbm, plane, idxv, outv):
    w = lax.axis_index("subcore") * 2 + lax.axis_index("core")

    @pl.loop(0, f * e // _NW)
    def _(pi):
      p = w + pi * _NW
      pf = p // e
      pe = p % e
      pltpu.sync_copy(t2T_hbm.at[pf, pe], plane)

      @pl.loop(0, b, step=_CH)
      def _(c):
        pltpu.sync_copy(idxT_hbm.at[pf, pl.ds(c, _CH)], idxv)

        @pl.loop(0, _CH, step=64)
        def _(j):
          for u in range(0, 64, 16):
            outv[pl.ds(j + u, 16)] = plsc.load_gather(
                plane, [idxv[pl.ds(j + u, 16)]])

        pltpu.sync_copy(outv, o2_hbm.at[pf, pe, pl.ds(c, _CH)])

    @pl.when(w < f)
    def _():
      pltpu.sync_copy(t1T_hbm.at[w, 0], plane)

      @pl.loop(0, b, step=_CH)
      def _(c):
        pltpu.sync_copy(idxT_hbm.at[w, pl.ds(c, _CH)], idxv)

        @pl.loop(0, _CH, step=64)
        def _(j):
          for u in range(0, 64, 16):
            outv[pl.ds(j + u, 16)] = plsc.load_gather(
                plane, [idxv[pl.ds(j + u, 16)]])

        pltpu.sync_copy(outv, o1_hbm.at[w, pl.ds(c, _CH)])

  return k(t2T, t1T, idxT)


def _stage_a_body(oT_ref, o1T_ref, dT_ref, wdl_ref, bdl_ref, w1d_ref, b1d_ref,
                  wl1_ref, bl1_ref, h1_ref, st_ref):
  i = pl.program_id(0)
  fe = oT_ref.shape[0]
  e = fe // o1T_ref.shape[0]
  oT = oT_ref[...]
  dT = dT_ref[...]
  sel = (lax.broadcasted_iota(jnp.int32, (e, fe), 0) ==
         lax.broadcasted_iota(jnp.int32, (e, fe), 1) % e).astype(jnp.float32)
  sum_embT = jax.lax.dot(sel, oT)
  order2 = 0.5 * (jnp.sum(sum_embT * sum_embT, axis=0, keepdims=True)
                  - jnp.sum(oT * oT, axis=0, keepdims=True))
  order1 = (jnp.sum(o1T_ref[...], axis=0, keepdims=True)
            + jax.lax.dot(w1d_ref[...], dT) + b1d_ref[...])
  ddT = jnp.maximum(jax.lax.dot(wdl_ref[...], dT) + bdl_ref[...], 0.0)
  dnnT = oT + ddT + order1 + order2
  h1 = jax.lax.dot(wl1_ref[...], dnnT) + bl1_ref[...]
  h1_ref[...] = h1
  st = jnp.concatenate([jnp.sum(h1, axis=1, keepdims=True),
                        jnp.sum(h1 * h1, axis=1, keepdims=True)], axis=1)

  @pl.when(i == 0)
  def _():
    st_ref[...] = st

  @pl.when(i > 0)
  def _():
    st_ref[...] = st_ref[...] + st


def _stage_mid_body(x_ref, st_in_ref, g_ref, be_ref, w_ref, bias_ref,
                    y_ref, st_ref, *, batch):
  i = pl.program_id(0)
  mean = st_in_ref[:, 0:1] / batch
  var = st_in_ref[:, 1:2] / batch - mean * mean
  a = g_ref[...] * lax.rsqrt(var + _EPS)
  c = be_ref[...] - mean * a
  x = jnp.maximum(x_ref[...] * a + c, 0.0)
  y = jax.lax.dot(w_ref[...], x) + bias_ref[...]
  y_ref[...] = y
  st = jnp.concatenate([jnp.sum(y, axis=1, keepdims=True),
                        jnp.sum(y * y, axis=1, keepdims=True)], axis=1)

  @pl.when(i == 0)
  def _():
    st_ref[...] = st

  @pl.when(i > 0)
  def _():
    st_ref[...] = st_ref[...] + st


def _stage_c_body(x_ref, st_in_ref, g_ref, be_ref, w_ref, bias_ref, out_ref,
                  *, batch):
  mean = st_in_ref[:, 0:1] / batch
  var = st_in_ref[:, 1:2] / batch - mean * mean
  a = g_ref[...] * lax.rsqrt(var + _EPS)
  c = be_ref[...] - mean * a
  x = jnp.maximum(x_ref[...] * a + c, 0.0)
  out_ref[...] = jax.lax.dot(w_ref[...], x) + bias_ref[...]


def _full(shape):
  return pl.BlockSpec(shape, lambda i: (0, 0))


def kernel(sparse, dense, order1_emb, order2_emb, W1d, b1d, Wdl, bdl, W_l1,
           b_l1, g1, be1, W_l2, b_l2, g2, be2, W_out, b_out):
  b, f = sparse.shape
  v, e = order2_emb.shape[1], order2_emb.shape[2]
  d = dense.shape[1]
  h1n, h2n = W_l1.shape[0], W_l2.shape[0]
  fe = f * e
  bb = 1024
  grid = (b // bb,)
  fl = jnp.float32

  o2T3, o1T = _sc_gather(jnp.transpose(order2_emb, (0, 2, 1)),
                         jnp.transpose(order1_emb, (0, 2, 1)), sparse.T)
  oT = o2T3.reshape(fe, b)

  h1T, st1 = pl.pallas_call(
      _stage_a_body,
      grid=grid,
      in_specs=[
          pl.BlockSpec((fe, bb), lambda i: (0, i)),
          pl.BlockSpec((f, bb), lambda i: (0, i)),
          pl.BlockSpec((d, bb), lambda i: (0, i)),
          _full((fe, d)), _full((fe, 1)), _full((1, d)), _full((1, 1)),
          _full((h1n, fe)), _full((h1n, 1)),
      ],
      out_specs=[pl.BlockSpec((h1n, bb), lambda i: (0, i)),
                 _full((h1n, 2))],
      out_shape=(jax.ShapeDtypeStruct((h1n, b), fl),
                 jax.ShapeDtypeStruct((h1n, 2), fl)),
  )(oT, o1T, dense.T, Wdl, bdl.reshape(fe, 1), W1d, b1d.reshape(1, 1),
    W_l1, b_l1.reshape(h1n, 1))

  h2T, st2 = pl.pallas_call(
      functools.partial(_stage_mid_body, batch=float(b)),
      grid=grid,
      in_specs=[
          pl.BlockSpec((h1n, bb), lambda i: (0, i)),
          _full((h1n, 2)), _full((h1n, 1)), _full((h1n, 1)),
          _full((h2n, h1n)), _full((h2n, 1)),
      ],
      out_specs=[pl.BlockSpec((h2n, bb), lambda i: (0, i)),
                 _full((h2n, 2))],
      out_shape=(jax.ShapeDtypeStruct((h2n, b), fl),
                 jax.ShapeDtypeStruct((h2n, 2), fl)),
  )(h1T, st1, g1.reshape(h1n, 1), be1.reshape(h1n, 1), W_l2,
    b_l2.reshape(h2n, 1))

  outT = pl.pallas_call(
      functools.partial(_stage_c_body, batch=float(b)),
      grid=grid,
      in_specs=[
          pl.BlockSpec((h2n, bb), lambda i: (0, i)),
          _full((h2n, 2)), _full((h2n, 1)), _full((h2n, 1)),
          _full((1, h2n)), _full((1, 1)),
      ],
      out_specs=pl.BlockSpec((1, bb), lambda i: (0, i)),
      out_shape=jax.ShapeDtypeStruct((1, b), fl),
  )(h2T, st2, g2.reshape(h2n, 1), be2.reshape(h2n, 1), W_out,
    b_out.reshape(1, 1))
  return outT.reshape(b, 1)


# speedup vs baseline: 1.3236x; 1.0544x over previous
import functools

import jax
import jax.numpy as jnp
from jax import lax
from jax.experimental import pallas as pl
from jax.experimental.pallas import tpu as pltpu
from jax.experimental.pallas import tpu_sc as plsc

_EPS = 1e-5
_NW = 32
_CH = 8192


def _sc_gather(t2T, t1T, idxT):
  f, e, v = t2T.shape
  b = idxT.shape[1]
  mesh = plsc.VectorSubcoreMesh(core_axis_name="core",
                                subcore_axis_name="subcore")

  @functools.partial(
      pl.kernel,
      out_type=(jax.ShapeDtypeStruct((f, e, b), jnp.float32),
                jax.ShapeDtypeStruct((f, b), jnp.float32)),
      mesh=mesh,
      scratch_types=[pltpu.VMEM((v,), jnp.float32),
                     pltpu.VMEM((_CH,), jnp.int32),
                     pltpu.VMEM((_CH,), jnp.float32)],
      compiler_params=pltpu.CompilerParams(use_tc_tiling_on_sc=True,
                                           needs_layout_passes=False))
  def k(t2T_hbm, t1T_hbm, idxT_hbm, o2_hbm, o1_hbm, plane, idxv, outv):
    w = lax.axis_index("subcore") * 2 + lax.axis_index("core")

    @pl.loop(0, f * e // _NW)
    def _(pi):
      p = w + pi * _NW
      pf = p // e
      pe = p % e
      pltpu.sync_copy(t2T_hbm.at[pf, pe], plane)

      @pl.loop(0, b, step=_CH)
      def _(c):
        pltpu.sync_copy(idxT_hbm.at[pf, pl.ds(c, _CH)], idxv)

        @pl.loop(0, _CH, step=64)
        def _(j):
          for u in range(0, 64, 16):
            outv[pl.ds(j + u, 16)] = plsc.load_gather(
                plane, [idxv[pl.ds(j + u, 16)]])

        pltpu.sync_copy(outv, o2_hbm.at[pf, pe, pl.ds(c, _CH)])

    @pl.when(w < f)
    def _():
      pltpu.sync_copy(t1T_hbm.at[w, 0], plane)

      @pl.loop(0, b, step=_CH)
      def _(c):
        pltpu.sync_copy(idxT_hbm.at[w, pl.ds(c, _CH)], idxv)

        @pl.loop(0, _CH, step=64)
        def _(j):
          for u in range(0, 64, 16):
            outv[pl.ds(j + u, 16)] = plsc.load_gather(
                plane, [idxv[pl.ds(j + u, 16)]])

        pltpu.sync_copy(outv, o1_hbm.at[w, pl.ds(c, _CH)])

  return k(t2T, t1T, idxT)


def _bn_coeffs(st_ref, g_ref, be_ref, batch):
  mean = st_ref[:, 0:1] / batch
  var = st_ref[:, 1:2] / batch - mean * mean
  a = g_ref[...] * lax.rsqrt(var + _EPS)
  c = be_ref[...] - mean * a
  return a, c


def _moments(y):
  return jnp.concatenate([jnp.sum(y, axis=1, keepdims=True),
                          jnp.sum(y * y, axis=1, keepdims=True)], axis=1)


def _fused_body(oT_ref, o1T_ref, dT_ref, wdl_ref, bdl_ref, w1d_ref, b1d_ref,
                wl1_ref, bl1_ref, g1_ref, be1_ref, wl2_ref, bl2_ref, g2_ref,
                be2_ref, wout_ref, bout_ref, out_ref, h1_s, h2_s, st1_s,
                st2_s, *, batch, bb):
  p = pl.program_id(0)
  i = pl.program_id(1)
  cols = pl.ds(i * bb, bb)

  @pl.when(p == 0)
  def _():
    fe = oT_ref.shape[0]
    e = fe // o1T_ref.shape[0]
    oT = oT_ref[...]
    dT = dT_ref[...]
    sel = (lax.broadcasted_iota(jnp.int32, (e, fe), 0) ==
           lax.broadcasted_iota(jnp.int32, (e, fe), 1) % e
           ).astype(jnp.float32)
    sum_embT = jax.lax.dot(sel, oT)
    order2 = 0.5 * (jnp.sum(sum_embT * sum_embT, axis=0, keepdims=True)
                    - jnp.sum(oT * oT, axis=0, keepdims=True))
    order1 = (jnp.sum(o1T_ref[...], axis=0, keepdims=True)
              + jax.lax.dot(w1d_ref[...], dT) + b1d_ref[...])
    ddT = jnp.maximum(jax.lax.dot(wdl_ref[...], dT) + bdl_ref[...], 0.0)
    dnnT = oT + ddT + order1 + order2
    h1 = jax.lax.dot(wl1_ref[...], dnnT) + bl1_ref[...]
    h1_s[:, cols] = h1
    st = _moments(h1)

    @pl.when(i == 0)
    def _():
      st1_s[...] = st

    @pl.when(i > 0)
    def _():
      st1_s[...] = st1_s[...] + st

  @pl.when(p == 1)
  def _():
    a, c = _bn_coeffs(st1_s, g1_ref, be1_ref, batch)
    x1 = jnp.maximum(h1_s[:, cols] * a + c, 0.0)
    h2 = jax.lax.dot(wl2_ref[...], x1) + bl2_ref[...]
    h2_s[:, cols] = h2
    st = _moments(h2)

    @pl.when(i == 0)
    def _():
      st2_s[...] = st

    @pl.when(i > 0)
    def _():
      st2_s[...] = st2_s[...] + st

  @pl.when(p == 2)
  def _():
    a, c = _bn_coeffs(st2_s, g2_ref, be2_ref, batch)
    x2 = jnp.maximum(h2_s[:, cols] * a + c, 0.0)
    out_ref[...] = jax.lax.dot(wout_ref[...], x2) + bout_ref[...]


def _full(shape):
  return pl.BlockSpec(shape, lambda p, i: (0, 0))


def kernel(sparse, dense, order1_emb, order2_emb, W1d, b1d, Wdl, bdl, W_l1,
           b_l1, g1, be1, W_l2, b_l2, g2, be2, W_out, b_out):
  b, f = sparse.shape
  v, e = order2_emb.shape[1], order2_emb.shape[2]
  d = dense.shape[1]
  h1n, h2n = W_l1.shape[0], W_l2.shape[0]
  fe = f * e
  bb = 1024
  fl = jnp.float32

  o2T3, o1T = _sc_gather(jnp.transpose(order2_emb, (0, 2, 1)),
                         jnp.transpose(order1_emb, (0, 2, 1)), sparse.T)
  oT = o2T3.reshape(fe, b)

  blk = lambda rows: pl.BlockSpec(
      (rows, bb), lambda p, i: (0, jnp.where(p == 0, i, 0)))
  outT = pl.pallas_call(
      functools.partial(_fused_body, batch=float(b), bb=bb),
      grid=(3, b // bb),
      in_specs=[
          blk(fe), blk(f), blk(d),
          _full((fe, d)), _full((fe, 1)), _full((1, d)), _full((1, 1)),
          _full((h1n, fe)), _full((h1n, 1)), _full((h1n, 1)),
          _full((h1n, 1)), _full((h2n, h1n)), _full((h2n, 1)),
          _full((h2n, 1)), _full((h2n, 1)), _full((1, h2n)), _full((1, 1)),
      ],
      out_specs=pl.BlockSpec((1, bb),
                             lambda p, i: (0, jnp.where(p == 2, i, 0))),
      out_shape=jax.ShapeDtypeStruct((1, b), fl),
      scratch_shapes=[pltpu.VMEM((h1n, b), fl), pltpu.VMEM((h2n, b), fl),
                      pltpu.VMEM((h1n, 2), fl), pltpu.VMEM((h2n, 2), fl)],
  )(oT, o1T, dense.T, Wdl, bdl.reshape(fe, 1), W1d, b1d.reshape(1, 1),
    W_l1, b_l1.reshape(h1n, 1), g1.reshape(h1n, 1), be1.reshape(h1n, 1),
    W_l2, b_l2.reshape(h2n, 1), g2.reshape(h2n, 1), be2.reshape(h2n, 1),
    W_out, b_out.reshape(1, 1))
  return outT.reshape(b, 1)


# speedup vs baseline: 1.5050x; 1.1371x over previous
import functools

import jax
import jax.numpy as jnp
from jax import lax
from jax.experimental import pallas as pl
from jax.experimental.pallas import tpu as pltpu
from jax.experimental.pallas import tpu_sc as plsc

_EPS = 1e-5
_NW = 32
_CH = 8192


def _sc_gather(t2T, t1T, idxT):
  f, e, v = t2T.shape
  b = idxT.shape[1]
  mesh = plsc.VectorSubcoreMesh(core_axis_name="core",
                                subcore_axis_name="subcore")

  @functools.partial(
      pl.kernel,
      out_type=(jax.ShapeDtypeStruct((f, e, b), jnp.float32),
                jax.ShapeDtypeStruct((f, b), jnp.float32)),
      mesh=mesh,
      scratch_types=[pltpu.VMEM((v,), jnp.float32),
                     pltpu.VMEM((b,), jnp.int32),
                     pltpu.VMEM((_CH,), jnp.float32)],
      compiler_params=pltpu.CompilerParams(use_tc_tiling_on_sc=True,
                                           needs_layout_passes=False))
  def k(t2T_hbm, t1T_hbm, idxT_hbm, o2_hbm, o1_hbm, plane, idxv, outv):
    w = lax.axis_index("subcore") * 2 + lax.axis_index("core")
    npw = f * e // _NW

    def gather_field(out_slice_fn):
      @pl.loop(0, b, step=_CH)
      def _(c):
        @pl.loop(0, _CH, step=128)
        def _(j):
          for u in range(0, 128, 16):
            outv[pl.ds(j + u, 16)] = plsc.load_gather(
                plane, [idxv[pl.ds(c + j + u, 16)]])

        pltpu.sync_copy(outv, out_slice_fn(c))

    @pl.loop(0, npw)
    def _(pi):
      p = w * npw + pi
      pf = p // e
      pe = p % e

      @pl.when((pi == 0) | (pf * e == p))
      def _():
        pltpu.sync_copy(idxT_hbm.at[pf], idxv)

      pltpu.sync_copy(t2T_hbm.at[pf, pe], plane)
      gather_field(lambda c, pf=pf, pe=pe: o2_hbm.at[pf, pe, pl.ds(c, _CH)])

    @pl.when(w < f)
    def _():
      pltpu.sync_copy(idxT_hbm.at[w], idxv)
      pltpu.sync_copy(t1T_hbm.at[w, 0], plane)
      gather_field(lambda c: o1_hbm.at[w, pl.ds(c, _CH)])

  return k(t2T, t1T, idxT)


def _bn_coeffs(st_ref, g_ref, be_ref, batch):
  mean = st_ref[:, 0:1] / batch
  var = st_ref[:, 1:2] / batch - mean * mean
  a = g_ref[...] * lax.rsqrt(var + _EPS)
  c = be_ref[...] - mean * a
  return a, c


def _moments(y):
  return jnp.concatenate([jnp.sum(y, axis=1, keepdims=True),
                          jnp.sum(y * y, axis=1, keepdims=True)], axis=1)


def _fused_body(oT_ref, o1T_ref, dT_ref, wdl_ref, bdl_ref, w1d_ref, b1d_ref,
                wl1_ref, bl1_ref, g1_ref, be1_ref, wl2_ref, bl2_ref, g2_ref,
                be2_ref, wout_ref, bout_ref, out_ref, h1_s, h2_s, st1_s,
                st2_s, *, batch, bb):
  p = pl.program_id(0)
  i = pl.program_id(1)
  cols = pl.ds(i * bb, bb)

  @pl.when(p == 0)
  def _():
    fe = oT_ref.shape[0]
    e = fe // o1T_ref.shape[0]
    oT = oT_ref[...]
    dT = dT_ref[...]
    sel = (lax.broadcasted_iota(jnp.int32, (e, fe), 0) ==
           lax.broadcasted_iota(jnp.int32, (e, fe), 1) % e
           ).astype(jnp.float32)
    sum_embT = jax.lax.dot(sel, oT)
    order2 = 0.5 * (jnp.sum(sum_embT * sum_embT, axis=0, keepdims=True)
                    - jnp.sum(oT * oT, axis=0, keepdims=True))
    order1 = (jnp.sum(o1T_ref[...], axis=0, keepdims=True)
              + jax.lax.dot(w1d_ref[...], dT) + b1d_ref[...])
    ddT = jnp.maximum(jax.lax.dot(wdl_ref[...], dT) + bdl_ref[...], 0.0)
    dnnT = oT + ddT + order1 + order2
    h1 = jax.lax.dot(wl1_ref[...], dnnT) + bl1_ref[...]
    h1_s[:, cols] = h1
    st = _moments(h1)

    @pl.when(i == 0)
    def _():
      st1_s[...] = st

    @pl.when(i > 0)
    def _():
      st1_s[...] = st1_s[...] + st

  @pl.when(p == 1)
  def _():
    a, c = _bn_coeffs(st1_s, g1_ref, be1_ref, batch)
    x1 = jnp.maximum(h1_s[:, cols] * a + c, 0.0)
    h2 = jax.lax.dot(wl2_ref[...], x1) + bl2_ref[...]
    h2_s[:, cols] = h2
    st = _moments(h2)

    @pl.when(i == 0)
    def _():
      st2_s[...] = st

    @pl.when(i > 0)
    def _():
      st2_s[...] = st2_s[...] + st

  @pl.when(p == 2)
  def _():
    a, c = _bn_coeffs(st2_s, g2_ref, be2_ref, batch)
    x2 = jnp.maximum(h2_s[:, cols] * a + c, 0.0)
    out_ref[...] = jax.lax.dot(wout_ref[...], x2) + bout_ref[...]


def _full(shape):
  return pl.BlockSpec(shape, lambda p, i: (0, 0))


def kernel(sparse, dense, order1_emb, order2_emb, W1d, b1d, Wdl, bdl, W_l1,
           b_l1, g1, be1, W_l2, b_l2, g2, be2, W_out, b_out):
  b, f = sparse.shape
  v, e = order2_emb.shape[1], order2_emb.shape[2]
  d = dense.shape[1]
  h1n, h2n = W_l1.shape[0], W_l2.shape[0]
  fe = f * e
  bb = 1024
  fl = jnp.float32

  o2T3, o1T = _sc_gather(jnp.transpose(order2_emb, (0, 2, 1)),
                         jnp.transpose(order1_emb, (0, 2, 1)), sparse.T)
  oT = o2T3.reshape(fe, b)

  blk = lambda rows: pl.BlockSpec(
      (rows, bb), lambda p, i: (0, jnp.where(p == 0, i, 0)))
  outT = pl.pallas_call(
      functools.partial(_fused_body, batch=float(b), bb=bb),
      grid=(3, b // bb),
      in_specs=[
          blk(fe), blk(f), blk(d),
          _full((fe, d)), _full((fe, 1)), _full((1, d)), _full((1, 1)),
          _full((h1n, fe)), _full((h1n, 1)), _full((h1n, 1)),
          _full((h1n, 1)), _full((h2n, h1n)), _full((h2n, 1)),
          _full((h2n, 1)), _full((h2n, 1)), _full((1, h2n)), _full((1, 1)),
      ],
      out_specs=pl.BlockSpec((1, bb),
                             lambda p, i: (0, jnp.where(p == 2, i, 0))),
      out_shape=jax.ShapeDtypeStruct((1, b), fl),
      scratch_shapes=[pltpu.VMEM((h1n, b), fl), pltpu.VMEM((h2n, b), fl),
                      pltpu.VMEM((h1n, 2), fl), pltpu.VMEM((h2n, 2), fl)],
  )(oT, o1T, dense.T, Wdl, bdl.reshape(fe, 1), W1d, b1d.reshape(1, 1),
    W_l1, b_l1.reshape(h1n, 1), g1.reshape(h1n, 1), be1.reshape(h1n, 1),
    W_l2, b_l2.reshape(h2n, 1), g2.reshape(h2n, 1), be2.reshape(h2n, 1),
    W_out, b_out.reshape(1, 1))
  return outT.reshape(b, 1)


# speedup vs baseline: 1.6037x; 1.0655x over previous
import functools

import jax
import jax.numpy as jnp
from jax import lax
from jax.experimental import pallas as pl
from jax.experimental.pallas import tpu as pltpu
from jax.experimental.pallas import tpu_sc as plsc

_EPS = 1e-5
_NW = 32
_CH = 8192


def _sc_gather(t2T, t1T, idxT):
  f, e, v = t2T.shape
  b = idxT.shape[1]
  mesh = plsc.VectorSubcoreMesh(core_axis_name="core",
                                subcore_axis_name="subcore")

  @functools.partial(
      pl.kernel,
      out_type=(jax.ShapeDtypeStruct((f, e, b), jnp.float32),
                jax.ShapeDtypeStruct((f, b), jnp.float32)),
      mesh=mesh,
      scratch_types=[pltpu.VMEM((v,), jnp.float32),
                     pltpu.VMEM((b,), jnp.int32),
                     pltpu.VMEM((_CH,), jnp.float32)],
      compiler_params=pltpu.CompilerParams(use_tc_tiling_on_sc=True,
                                           needs_layout_passes=False))
  def k(t2T_hbm, t1T_hbm, idxT_hbm, o2_hbm, o1_hbm, plane, idxv, outv):
    w = lax.axis_index("subcore") * 2 + lax.axis_index("core")
    npw = f * e // _NW

    def gather_field(out_slice_fn):
      @pl.loop(0, b, step=_CH)
      def _(c):
        @pl.loop(0, _CH, step=128)
        def _(j):
          for u in range(0, 128, 16):
            outv[pl.ds(j + u, 16)] = plsc.load_gather(
                plane, [idxv[pl.ds(c + j + u, 16)]])

        pltpu.sync_copy(outv, out_slice_fn(c))

    @pl.loop(0, npw)
    def _(pi):
      p = w * npw + pi
      pf = p // e
      pe = p % e

      @pl.when((pi == 0) | (pf * e == p))
      def _():
        pltpu.sync_copy(idxT_hbm.at[pf], idxv)

      pltpu.sync_copy(t2T_hbm.at[pf, pe], plane)
      gather_field(lambda c, pf=pf, pe=pe: o2_hbm.at[pf, pe, pl.ds(c, _CH)])

    @pl.when(w < f)
    def _():
      pltpu.sync_copy(idxT_hbm.at[w], idxv)
      pltpu.sync_copy(t1T_hbm.at[w, 0], plane)
      gather_field(lambda c: o1_hbm.at[w, pl.ds(c, _CH)])

  return k(t2T, t1T, idxT)


def _bn_coeffs(st_ref, g_ref, be_ref, batch):
  mean = st_ref[:, 0:1] / batch
  var = st_ref[:, 1:2] / batch - mean * mean
  a = g_ref[...] * lax.rsqrt(var + _EPS)
  c = be_ref[...] - mean * a
  return a, c


def _moments(y):
  return jnp.concatenate([jnp.sum(y, axis=1, keepdims=True),
                          jnp.sum(y * y, axis=1, keepdims=True)], axis=1)


def _fused_body(oT_ref, o1T_ref, dT_ref, wdl_ref, bdl_ref, w1d_ref, b1d_ref,
                wl1_ref, bl1_ref, g1_ref, be1_ref, wl2_ref, bl2_ref, g2_ref,
                be2_ref, wout_ref, bout_ref, out_ref, h1_s, h2_s, st1_s,
                st2_s, *, batch, bb):
  p = pl.program_id(0)
  i = pl.program_id(1)
  cols = pl.ds(i * bb, bb)

  @pl.when(p == 0)
  def _():
    fe = oT_ref.shape[0]
    e = fe // o1T_ref.shape[0]
    oT = oT_ref[...]
    dT = dT_ref[...]
    sel = (lax.broadcasted_iota(jnp.int32, (e, fe), 0) ==
           lax.broadcasted_iota(jnp.int32, (e, fe), 1) % e
           ).astype(jnp.float32)
    sum_embT = jax.lax.dot(sel, oT)
    order2 = 0.5 * (jnp.sum(sum_embT * sum_embT, axis=0, keepdims=True)
                    - jnp.sum(oT * oT, axis=0, keepdims=True))
    order1 = (jnp.sum(o1T_ref[...], axis=0, keepdims=True)
              + jax.lax.dot(w1d_ref[...], dT) + b1d_ref[...])
    ddT = jnp.maximum(jax.lax.dot(wdl_ref[...], dT) + bdl_ref[...], 0.0)
    dnnT = oT + ddT + order1 + order2
    h1 = jax.lax.dot(wl1_ref[...], dnnT) + bl1_ref[...]
    h1_s[:, cols] = h1
    st = _moments(h1)

    @pl.when(i == 0)
    def _():
      st1_s[...] = st

    @pl.when(i > 0)
    def _():
      st1_s[...] = st1_s[...] + st

  @pl.when(p == 1)
  def _():
    a, c = _bn_coeffs(st1_s, g1_ref, be1_ref, batch)
    x1 = jnp.maximum(h1_s[:, cols] * a + c, 0.0)
    h2 = jax.lax.dot(wl2_ref[...], x1) + bl2_ref[...]
    h2_s[:, cols] = h2
    st = _moments(h2)

    @pl.when(i == 0)
    def _():
      st2_s[...] = st

    @pl.when(i > 0)
    def _():
      st2_s[...] = st2_s[...] + st

  @pl.when(p == 2)
  def _():
    a, c = _bn_coeffs(st2_s, g2_ref, be2_ref, batch)
    x2 = jnp.maximum(h2_s[:, cols] * a + c, 0.0)
    out_ref[...] = jax.lax.dot(wout_ref[...], x2) + bout_ref[...]


def _full(shape):
  return pl.BlockSpec(shape, lambda p, i: (0, 0))


def kernel(sparse, dense, order1_emb, order2_emb, W1d, b1d, Wdl, bdl, W_l1,
           b_l1, g1, be1, W_l2, b_l2, g2, be2, W_out, b_out):
  b, f = sparse.shape
  v, e = order2_emb.shape[1], order2_emb.shape[2]
  d = dense.shape[1]
  h1n, h2n = W_l1.shape[0], W_l2.shape[0]
  fe = f * e
  bb = 2048
  fl = jnp.float32

  o2T3, o1T = _sc_gather(jnp.transpose(order2_emb, (0, 2, 1)),
                         jnp.transpose(order1_emb, (0, 2, 1)), sparse.T)
  oT = o2T3.reshape(fe, b)

  blk = lambda rows: pl.BlockSpec(
      (rows, bb), lambda p, i: (0, jnp.where(p == 0, i, 0)))
  outT = pl.pallas_call(
      functools.partial(_fused_body, batch=float(b), bb=bb),
      grid=(3, b // bb),
      in_specs=[
          blk(fe), blk(f), blk(d),
          _full((fe, d)), _full((fe, 1)), _full((1, d)), _full((1, 1)),
          _full((h1n, fe)), _full((h1n, 1)), _full((h1n, 1)),
          _full((h1n, 1)), _full((h2n, h1n)), _full((h2n, 1)),
          _full((h2n, 1)), _full((h2n, 1)), _full((1, h2n)), _full((1, 1)),
      ],
      out_specs=pl.BlockSpec((1, bb),
                             lambda p, i: (0, jnp.where(p == 2, i, 0))),
      out_shape=jax.ShapeDtypeStruct((1, b), fl),
      scratch_shapes=[pltpu.VMEM((h1n, b), fl), pltpu.VMEM((h2n, b), fl),
                      pltpu.VMEM((h1n, 2), fl), pltpu.VMEM((h2n, 2), fl)],
  )(oT, o1T, dense.T, Wdl, bdl.reshape(fe, 1), W1d, b1d.reshape(1, 1),
    W_l1, b_l1.reshape(h1n, 1), g1.reshape(h1n, 1), be1.reshape(h1n, 1),
    W_l2, b_l2.reshape(h2n, 1), g2.reshape(h2n, 1), be2.reshape(h2n, 1),
    W_out, b_out.reshape(1, 1))
  return outT.reshape(b, 1)


# speedup vs baseline: 1.6504x; 1.0292x over previous
import functools

import jax
import jax.numpy as jnp
from jax import lax
from jax.experimental import pallas as pl
from jax.experimental.pallas import tpu as pltpu
from jax.experimental.pallas import tpu_sc as plsc

_EPS = 1e-5
_NW = 32
_CH = 8192


def _sc_gather(t2T, t1T, idxT):
  f, e, v = t2T.shape
  b = idxT.shape[1]
  mesh = plsc.VectorSubcoreMesh(core_axis_name="core",
                                subcore_axis_name="subcore")

  @functools.partial(
      pl.kernel,
      out_type=(jax.ShapeDtypeStruct((f, e, b), jnp.float32),
                jax.ShapeDtypeStruct((f, b), jnp.float32)),
      mesh=mesh,
      scratch_types=[pltpu.VMEM((v,), jnp.float32),
                     pltpu.VMEM((b,), jnp.int32),
                     pltpu.VMEM((_CH,), jnp.float32)],
      compiler_params=pltpu.CompilerParams(use_tc_tiling_on_sc=True,
                                           needs_layout_passes=False))
  def k(t2T_hbm, t1T_hbm, idxT_hbm, o2_hbm, o1_hbm, plane, idxv, outv):
    w = lax.axis_index("subcore") * 2 + lax.axis_index("core")
    npw = f * e // _NW

    def gather_field(out_slice_fn):
      @pl.loop(0, b, step=_CH)
      def _(c):
        @pl.loop(0, _CH, step=128)
        def _(j):
          for u in range(0, 128, 16):
            outv[pl.ds(j + u, 16)] = plsc.load_gather(
                plane, [idxv[pl.ds(c + j + u, 16)]])

        pltpu.sync_copy(outv, out_slice_fn(c))

    @pl.loop(0, npw)
    def _(pi):
      p = w * npw + pi
      pf = p // e
      pe = p % e

      @pl.when((pi == 0) | (pf * e == p))
      def _():
        pltpu.sync_copy(idxT_hbm.at[pf], idxv)

      pltpu.sync_copy(t2T_hbm.at[pf, pe], plane)
      gather_field(lambda c, pf=pf, pe=pe: o2_hbm.at[pf, pe, pl.ds(c, _CH)])

    @pl.when(w < f)
    def _():
      pltpu.sync_copy(idxT_hbm.at[w], idxv)
      pltpu.sync_copy(t1T_hbm.at[w, 0], plane)
      gather_field(lambda c: o1_hbm.at[w, pl.ds(c, _CH)])

  return k(t2T, t1T, idxT)


def _bn_coeffs(st_ref, g_ref, be_ref, batch):
  mean = st_ref[:, 0:1] / batch
  var = st_ref[:, 1:2] / batch - mean * mean
  a = g_ref[...] * lax.rsqrt(var + _EPS)
  c = be_ref[...] - mean * a
  return a, c


def _moments(y):
  return jnp.concatenate([jnp.sum(y, axis=1, keepdims=True),
                          jnp.sum(y * y, axis=1, keepdims=True)], axis=1)


def _fused_body(oT_ref, o1T_ref, dT_ref, wdl_ref, bdl_ref, w1d_ref, b1d_ref,
                wl1_ref, bl1_ref, g1_ref, be1_ref, wl2_ref, bl2_ref, g2_ref,
                be2_ref, wout_ref, bout_ref, out_ref, h1_s, h2_s, st1_s,
                st2_s, *, batch, bb):
  p = pl.program_id(0)
  i = pl.program_id(1)
  cols = pl.ds(i * bb, bb)

  @pl.when(p == 0)
  def _():
    fe = oT_ref.shape[0]
    e = fe // o1T_ref.shape[0]
    oT = oT_ref[...]
    dT = dT_ref[...]
    sel = (lax.broadcasted_iota(jnp.int32, (e, fe), 0) ==
           lax.broadcasted_iota(jnp.int32, (e, fe), 1) % e
           ).astype(jnp.float32)
    sum_embT = jax.lax.dot(sel, oT)
    order2 = 0.5 * (jnp.sum(sum_embT * sum_embT, axis=0, keepdims=True)
                    - jnp.sum(oT * oT, axis=0, keepdims=True))
    order1 = (jnp.sum(o1T_ref[...], axis=0, keepdims=True)
              + jax.lax.dot(w1d_ref[...], dT) + b1d_ref[...])
    ddT = jnp.maximum(jax.lax.dot(wdl_ref[...], dT) + bdl_ref[...], 0.0)
    dnnT = oT + ddT + order1 + order2
    h1 = jax.lax.dot(wl1_ref[...], dnnT) + bl1_ref[...]
    h1_s[:, cols] = h1
    st = _moments(h1)

    @pl.when(i == 0)
    def _():
      st1_s[...] = st

    @pl.when(i > 0)
    def _():
      st1_s[...] = st1_s[...] + st

  @pl.when(p == 1)
  def _():
    a, c = _bn_coeffs(st1_s, g1_ref, be1_ref, batch)
    x1 = jnp.maximum(h1_s[:, cols] * a + c, 0.0)
    h2 = jax.lax.dot(wl2_ref[...], x1) + bl2_ref[...]
    h2_s[:, cols] = h2
    st = _moments(h2)

    @pl.when(i == 0)
    def _():
      st2_s[...] = st

    @pl.when(i > 0)
    def _():
      st2_s[...] = st2_s[...] + st

  @pl.when(p == 2)
  def _():
    a, c = _bn_coeffs(st2_s, g2_ref, be2_ref, batch)
    x2 = jnp.maximum(h2_s[:, cols] * a + c, 0.0)
    out_ref[...] = jax.lax.dot(wout_ref[...], x2) + bout_ref[...]


def _full(shape):
  return pl.BlockSpec(shape, lambda p, i: (0, 0))


def kernel(sparse, dense, order1_emb, order2_emb, W1d, b1d, Wdl, bdl, W_l1,
           b_l1, g1, be1, W_l2, b_l2, g2, be2, W_out, b_out):
  b, f = sparse.shape
  v, e = order2_emb.shape[1], order2_emb.shape[2]
  d = dense.shape[1]
  h1n, h2n = W_l1.shape[0], W_l2.shape[0]
  fe = f * e
  bb = 4096
  fl = jnp.float32

  o2T3, o1T = _sc_gather(jnp.transpose(order2_emb, (0, 2, 1)),
                         jnp.transpose(order1_emb, (0, 2, 1)), sparse.T)
  oT = o2T3.reshape(fe, b)

  blk = lambda rows: pl.BlockSpec(
      (rows, bb), lambda p, i: (0, jnp.where(p == 0, i, 0)))
  outT = pl.pallas_call(
      functools.partial(_fused_body, batch=float(b), bb=bb),
      grid=(3, b // bb),
      in_specs=[
          blk(fe), blk(f), blk(d),
          _full((fe, d)), _full((fe, 1)), _full((1, d)), _full((1, 1)),
          _full((h1n, fe)), _full((h1n, 1)), _full((h1n, 1)),
          _full((h1n, 1)), _full((h2n, h1n)), _full((h2n, 1)),
          _full((h2n, 1)), _full((h2n, 1)), _full((1, h2n)), _full((1, 1)),
      ],
      out_specs=pl.BlockSpec((1, bb),
                             lambda p, i: (0, jnp.where(p == 2, i, 0))),
      out_shape=jax.ShapeDtypeStruct((1, b), fl),
      scratch_shapes=[pltpu.VMEM((h1n, b), fl), pltpu.VMEM((h2n, b), fl),
                      pltpu.VMEM((h1n, 2), fl), pltpu.VMEM((h2n, 2), fl)],
  )(oT, o1T, dense.T, Wdl, bdl.reshape(fe, 1), W1d, b1d.reshape(1, 1),
    W_l1, b_l1.reshape(h1n, 1), g1.reshape(h1n, 1), be1.reshape(h1n, 1),
    W_l2, b_l2.reshape(h2n, 1), g2.reshape(h2n, 1), be2.reshape(h2n, 1),
    W_out, b_out.reshape(1, 1))
  return outT.reshape(b, 1)


# speedup vs baseline: 1.6531x; 1.0016x over previous
import functools

import jax
import jax.numpy as jnp
from jax import lax
from jax.experimental import pallas as pl
from jax.experimental.pallas import tpu as pltpu
from jax.experimental.pallas import tpu_sc as plsc

_EPS = 1e-5
_NW = 32
_CH = 8192


def _sc_gather(t2T, t1T, idxT):
  f, e, v = t2T.shape
  b = idxT.shape[1]
  mesh = plsc.VectorSubcoreMesh(core_axis_name="core",
                                subcore_axis_name="subcore")

  @functools.partial(
      pl.kernel,
      out_type=(jax.ShapeDtypeStruct((f, e, b), jnp.float32),
                jax.ShapeDtypeStruct((f, b), jnp.float32)),
      mesh=mesh,
      scratch_types=[pltpu.VMEM((v,), jnp.float32),
                     pltpu.VMEM((b,), jnp.int32),
                     pltpu.VMEM((_CH,), jnp.float32)],
      compiler_params=pltpu.CompilerParams(use_tc_tiling_on_sc=True,
                                           needs_layout_passes=False))
  def k(t2T_hbm, t1T_hbm, idxT_hbm, o2_hbm, o1_hbm, plane, idxv, outv):
    w = lax.axis_index("subcore") * 2 + lax.axis_index("core")
    npw = f * e // _NW

    def gather_field(out_slice_fn):
      @pl.loop(0, b, step=_CH)
      def _(c):
        @pl.loop(0, _CH, step=128)
        def _(j):
          for u in range(0, 128, 16):
            outv[pl.ds(j + u, 16)] = plsc.load_gather(
                plane, [idxv[pl.ds(c + j + u, 16)]])

        pltpu.sync_copy(outv, out_slice_fn(c))

    @pl.loop(0, npw)
    def _(pi):
      p = w * npw + pi
      pf = p // e
      pe = p % e

      @pl.when((pi == 0) | (pf * e == p))
      def _():
        pltpu.sync_copy(idxT_hbm.at[pf], idxv)

      pltpu.sync_copy(t2T_hbm.at[pf, pe], plane)
      gather_field(lambda c, pf=pf, pe=pe: o2_hbm.at[pf, pe, pl.ds(c, _CH)])

    @pl.when(w < f)
    def _():
      pltpu.sync_copy(idxT_hbm.at[w], idxv)
      pltpu.sync_copy(t1T_hbm.at[w, 0], plane)
      gather_field(lambda c: o1_hbm.at[w, pl.ds(c, _CH)])

  return k(t2T, t1T, idxT)


def _bn_coeffs(st_ref, g_ref, be_ref, batch):
  mean = st_ref[:, 0:1] / batch
  var = st_ref[:, 1:2] / batch - mean * mean
  a = g_ref[...] * lax.rsqrt(var + _EPS)
  c = be_ref[...] - mean * a
  return a, c


def _moments(y):
  return jnp.concatenate([jnp.sum(y, axis=1, keepdims=True),
                          jnp.sum(y * y, axis=1, keepdims=True)], axis=1)


def _fused_body(oT_ref, o1T_ref, dT_ref, wdl_ref, bdl_ref, w1d_ref, b1d_ref,
                wl1_ref, bl1_ref, g1_ref, be1_ref, wl2_ref, bl2_ref, g2_ref,
                be2_ref, wout_ref, bout_ref, out_ref, h1_s, h2_s, st1_s,
                st2_s, *, batch, bb):
  p = pl.program_id(0)
  i = pl.program_id(1)
  cols = pl.ds(i * bb, bb)

  @pl.when(p == 0)
  def _():
    fe = oT_ref.shape[0]
    e = fe // o1T_ref.shape[0]
    oT = oT_ref[...]
    dT = dT_ref[...]
    sel = (lax.broadcasted_iota(jnp.int32, (e, fe), 0) ==
           lax.broadcasted_iota(jnp.int32, (e, fe), 1) % e
           ).astype(jnp.float32)
    sum_embT = jax.lax.dot(sel, oT)
    order2 = 0.5 * (jnp.sum(sum_embT * sum_embT, axis=0, keepdims=True)
                    - jnp.sum(oT * oT, axis=0, keepdims=True))
    order1 = (jnp.sum(o1T_ref[...], axis=0, keepdims=True)
              + jax.lax.dot(w1d_ref[...], dT) + b1d_ref[...])
    ddT = jnp.maximum(jax.lax.dot(wdl_ref[...], dT) + bdl_ref[...], 0.0)
    dnnT = oT + ddT + order1 + order2
    h1 = jax.lax.dot(wl1_ref[...], dnnT) + bl1_ref[...]
    h1_s[:, cols] = h1
    st = _moments(h1)

    @pl.when(i == 0)
    def _():
      st1_s[...] = st

    @pl.when(i > 0)
    def _():
      st1_s[...] = st1_s[...] + st

  @pl.when(p == 1)
  def _():
    a, c = _bn_coeffs(st1_s, g1_ref, be1_ref, batch)
    x1 = jnp.maximum(h1_s[:, cols] * a + c, 0.0)
    h2 = jax.lax.dot(wl2_ref[...], x1) + bl2_ref[...]
    h2_s[:, cols] = h2
    st = _moments(h2)

    @pl.when(i == 0)
    def _():
      st2_s[...] = st

    @pl.when(i > 0)
    def _():
      st2_s[...] = st2_s[...] + st

  @pl.when(p == 2)
  def _():
    a, c = _bn_coeffs(st2_s, g2_ref, be2_ref, batch)
    x2 = jnp.maximum(h2_s[:, cols] * a + c, 0.0)
    out_ref[...] = jax.lax.dot(wout_ref[...], x2) + bout_ref[...]


def _full(shape):
  return pl.BlockSpec(shape, lambda p, i: (0, 0))


def kernel(sparse, dense, order1_emb, order2_emb, W1d, b1d, Wdl, bdl, W_l1,
           b_l1, g1, be1, W_l2, b_l2, g2, be2, W_out, b_out):
  b, f = sparse.shape
  v, e = order2_emb.shape[1], order2_emb.shape[2]
  d = dense.shape[1]
  h1n, h2n = W_l1.shape[0], W_l2.shape[0]
  fe = f * e
  bb = 8192
  fl = jnp.float32

  o2T3, o1T = _sc_gather(jnp.transpose(order2_emb, (0, 2, 1)),
                         jnp.transpose(order1_emb, (0, 2, 1)), sparse.T)
  oT = o2T3.reshape(fe, b)

  blk = lambda rows: pl.BlockSpec(
      (rows, bb), lambda p, i: (0, jnp.where(p == 0, i, 0)))
  outT = pl.pallas_call(
      functools.partial(_fused_body, batch=float(b), bb=bb),
      grid=(3, b // bb),
      in_specs=[
          blk(fe), blk(f), blk(d),
          _full((fe, d)), _full((fe, 1)), _full((1, d)), _full((1, 1)),
          _full((h1n, fe)), _full((h1n, 1)), _full((h1n, 1)),
          _full((h1n, 1)), _full((h2n, h1n)), _full((h2n, 1)),
          _full((h2n, 1)), _full((h2n, 1)), _full((1, h2n)), _full((1, 1)),
      ],
      out_specs=pl.BlockSpec((1, bb),
                             lambda p, i: (0, jnp.where(p == 2, i, 0))),
      out_shape=jax.ShapeDtypeStruct((1, b), fl),
      scratch_shapes=[pltpu.VMEM((h1n, b), fl), pltpu.VMEM((h2n, b), fl),
                      pltpu.VMEM((h1n, 2), fl), pltpu.VMEM((h2n, 2), fl)],
  )(oT, o1T, dense.T, Wdl, bdl.reshape(fe, 1), W1d, b1d.reshape(1, 1),
    W_l1, b_l1.reshape(h1n, 1), g1.reshape(h1n, 1), be1.reshape(h1n, 1),
    W_l2, b_l2.reshape(h2n, 1), g2.reshape(h2n, 1), be2.reshape(h2n, 1),
    W_out, b_out.reshape(1, 1))
  return outT.reshape(b, 1)
